# Initial kernel scaffold; baseline (speedup 1.0000x reference)
import jax
import jax.numpy as jnp
from jax.experimental import pallas as pl


def kernel(x, edge_index, edge_attr, batch, params):
    raise NotImplementedError("write your pallas kernel here")


# speedup vs baseline: 1.8888x; 1.8888x over previous
import functools
import jax
import jax.numpy as jnp
from jax import lax
from jax.experimental import pallas as pl
from jax.experimental.pallas import tpu as pltpu
from jax.experimental.pallas import tpu_sc as plsc

N = 100000
E = 3200000
NG = 64
HID = 16
RNN = 32
JK = 48

NC = 2
NS = 16
NW = NC * NS
EPW = E // NW
NP = 100096
RPS = NP // NS
NPH = 8
QR = 12500
QNP = 12544
RPH = QNP // NS
JUNK = 12520

BA = 4000
BB = 800
RBLK = 2000
EBLK = 16000

_mesh = plsc.VectorSubcoreMesh(core_axis_name="c", subcore_axis_name="s")


@functools.partial(
    pl.kernel, mesh=_mesh,
    out_type=jax.ShapeDtypeStruct((E,), jnp.float32),
    scratch_types=[
        pltpu.VMEM((N,), jnp.float32),
        pltpu.VMEM((BA,), jnp.int32),
        pltpu.VMEM((BA,), jnp.float32),
        pltpu.VMEM((BA,), jnp.float32),
    ],
    compiler_params=pltpu.CompilerParams(needs_layout_passes=False),
)
def _sc_src_score(src_hbm, esc_hbm, stab_hbm, t_hbm, tab_v, idx_v, esc_v, t_v):
    c = lax.axis_index("c")
    s = lax.axis_index("s")
    wid = s * NC + c
    base = wid * EPW
    pltpu.sync_copy(stab_hbm, tab_v)

    def blk(b, carry):
        off = base + b * BA
        pltpu.sync_copy(src_hbm.at[pl.ds(off, BA)], idx_v)
        pltpu.sync_copy(esc_hbm.at[pl.ds(off, BA)], esc_v)

        def grp(i, cc):
            ii = i * 16
            idx16 = idx_v[pl.ds(ii, 16)]
            v = plsc.load_gather(tab_v, [idx16])
            t_v[pl.ds(ii, 16)] = v + esc_v[pl.ds(ii, 16)]
            return cc

        lax.fori_loop(0, BA // 16, grp, 0)
        pltpu.sync_copy(t_v, t_hbm.at[pl.ds(off, BA)])
        return carry

    lax.fori_loop(0, EPW // BA, blk, 0)


@functools.partial(
    pl.kernel, mesh=_mesh,
    out_type=(jax.ShapeDtypeStruct((NC, NPH, QNP, HID), jnp.float32),
              jax.ShapeDtypeStruct((NC * NPH * QNP,), jnp.float32)),
    scratch_types=[
        pltpu.VMEM((N,), jnp.float32),
        pltpu.VMEM((BB,), jnp.int32),
        pltpu.VMEM((BB,), jnp.int32),
        pltpu.VMEM((BB,), jnp.float32),
        pltpu.VMEM((BB,), jnp.float32),
        pltpu.VMEM((BB, HID), jnp.float32),
        pltpu.VMEM_SHARED((QNP, HID), jnp.float32),
        pltpu.VMEM_SHARED((QNP,), jnp.float32),
        pltpu.SemaphoreType.DMA,
    ],
    compiler_params=pltpu.CompilerParams(needs_layout_passes=False,
                                         use_tc_tiling_on_sc=False),
)
def _sc_message(src_hbm, dst_hbm, t_hbm, dtab_hbm, hs_hbm, outm_hbm, outd_hbm,
                tab_v, src_v, dst_v, t_v, ex_v, rows_v, accm, accd, sem):
    c = lax.axis_index("c")
    s = lax.axis_index("s")
    wid = s * NC + c
    base = wid * EPW
    r0 = s * RPH
    pltpu.sync_copy(dtab_hbm, tab_v)

    for p in range(NPH):
        lo = p * QR

        def z16(i, cc):
            rows_v[i, :] = jnp.zeros((16,), jnp.float32)
            return cc

        lax.fori_loop(0, BB, z16, 0)

        def z1(i, cc):
            ex_v[pl.ds(i * 16, 16)] = jnp.zeros((16,), jnp.float32)
            return cc

        lax.fori_loop(0, BB // 16, z1, 0)

        done = 0
        while done < RPH:
            step = min(BB, RPH - done)
            pltpu.sync_copy(rows_v.at[pl.ds(0, step)],
                            accm.at[pl.ds(r0 + done, step)])
            pltpu.sync_copy(ex_v.at[pl.ds(0, step)],
                            accd.at[pl.ds(r0 + done, step)])
            done += step
        plsc.subcore_barrier()

        def blk(b, carry):
            off = base + b * BB
            pltpu.sync_copy(src_hbm.at[pl.ds(off, BB)], src_v)
            pltpu.sync_copy(dst_hbm.at[pl.ds(off, BB)], dst_v)
            pltpu.sync_copy(t_hbm.at[pl.ds(off, BB)], t_v)
            gat = pltpu.async_copy(hs_hbm.at[src_v], rows_v, sem)

            def grp(i, cc):
                ii = i * 16
                d16 = dst_v[pl.ds(ii, 16)]
                sd = plsc.load_gather(tab_v, [d16])
                a = t_v[pl.ds(ii, 16)] + sd
                a = jnp.where(a > 0.0, a, 0.2 * a)
                ex_v[pl.ds(ii, 16)] = jnp.exp(a)
                lid = d16 - lo
                inr = (lid >= 0) & (lid < QR)
                dst_v[pl.ds(ii, 16)] = jnp.where(inr, lid, JUNK)
                return cc

            lax.fori_loop(0, BB // 16, grp, 0)
            gat.wait()

            def scale(j, cc):
                jj = j * 16
                ex16 = ex_v[pl.ds(jj, 16)]
                for u in range(16):
                    rows_v[jj + u, :] = rows_v[jj + u, :] * ex16[u]
                return cc

            lax.fori_loop(0, BB // 16, scale, 0)
            pltpu.sync_copy(rows_v, accm.at[dst_v], add=True)
            pltpu.sync_copy(ex_v, accd.at[dst_v], add=True)
            return carry

        lax.fori_loop(0, EPW // BB, blk, 0)
        plsc.subcore_barrier()
        pltpu.sync_copy(accm.at[pl.ds(r0, RPH)],
                        outm_hbm.at[c, p, pl.ds(r0, RPH)])
        pltpu.sync_copy(accd.at[pl.ds(r0, RPH)],
                        outd_hbm.at[pl.ds((c * NPH + p) * QNP + r0, RPH)])


def _proj0_body(x_ref, w_ref, b_ref, a_ref, hs_ref, s2_ref):
    hs = jnp.dot(x_ref[...], w_ref[...], preferred_element_type=jnp.float32)
    hs = hs + b_ref[...]
    hs_ref[...] = hs
    s2_ref[...] = jnp.dot(hs, a_ref[...], preferred_element_type=jnp.float32)


def _tc_proj0(x, w, b, a):
    return pl.pallas_call(
        _proj0_body,
        grid=(N // RBLK,),
        in_specs=[
            pl.BlockSpec((RBLK, 128), lambda i: (i, 0)),
            pl.BlockSpec((128, HID), lambda i: (0, 0)),
            pl.BlockSpec((1, HID), lambda i: (0, 0)),
            pl.BlockSpec((HID, 2), lambda i: (0, 0)),
        ],
        out_specs=[
            pl.BlockSpec((RBLK, HID), lambda i: (i, 0)),
            pl.BlockSpec((RBLK, 2), lambda i: (i, 0)),
        ],
        out_shape=[
            jax.ShapeDtypeStruct((N, HID), jnp.float32),
            jax.ShapeDtypeStruct((N, 2), jnp.float32),
        ],
    )(x, w, b, a)


def _escore_body(ea_ref, w_ref, c_ref, o_ref):
    o_ref[...] = jnp.dot(ea_ref[...], w_ref[...],
                         preferred_element_type=jnp.float32) + c_ref[...]


def _tc_escore(ea, w, c):
    return pl.pallas_call(
        _escore_body,
        grid=(E // EBLK,),
        in_specs=[
            pl.BlockSpec((EBLK, 16), lambda i: (i, 0)),
            pl.BlockSpec((16, 3), lambda i: (0, 0)),
            pl.BlockSpec((1, 3), lambda i: (0, 0)),
        ],
        out_specs=pl.BlockSpec((EBLK, 3), lambda i: (i, 0)),
        out_shape=jax.ShapeDtypeStruct((E, 3), jnp.float32),
    )(ea, w, c)


def _norm_body(mm_ref, md_ref, bias_ref, lin_ref, a_ref, h_ref, hs_ref, s2_ref):
    m = mm_ref[0] + mm_ref[1]
    d = md_ref[0] + md_ref[1]
    h = m / (d + 1e-16) + bias_ref[...]
    h = jnp.where(h > 0.0, h, 0.01 * h)
    h_ref[...] = h
    hs = jnp.dot(h, lin_ref[...], preferred_element_type=jnp.float32)
    hs_ref[...] = hs
    s2_ref[...] = jnp.dot(hs, a_ref[...], preferred_element_type=jnp.float32)


def _tc_norm(mm, md, bias, lin, a):
    md3 = md.reshape(NC, N, 1)
    return pl.pallas_call(
        _norm_body,
        grid=(N // RBLK,),
        in_specs=[
            pl.BlockSpec((NC, RBLK, HID), lambda i: (0, i, 0)),
            pl.BlockSpec((NC, RBLK, 1), lambda i: (0, i, 0)),
            pl.BlockSpec((1, HID), lambda i: (0, 0)),
            pl.BlockSpec((HID, HID), lambda i: (0, 0)),
            pl.BlockSpec((HID, 2), lambda i: (0, 0)),
        ],
        out_specs=[
            pl.BlockSpec((RBLK, HID), lambda i: (i, 0)),
            pl.BlockSpec((RBLK, HID), lambda i: (i, 0)),
            pl.BlockSpec((RBLK, 2), lambda i: (i, 0)),
        ],
        out_shape=[
            jax.ShapeDtypeStruct((N, HID), jnp.float32),
            jax.ShapeDtypeStruct((N, HID), jnp.float32),
            jax.ShapeDtypeStruct((N, 2), jnp.float32),
        ],
    )(mm, md3, bias, lin, a)


def _gru_body(starts_ref, counts_ref, tm_ref, cvec_ref, hjk_ref,
              wif_ref, whf_ref, wib_ref, whb_ref, y_ref, xt_ref):
    tmax = tm_ref[0, 0]

    def run_dir(wi_ref, wh_ref, lane0, reverse):
        wi = wi_ref[...]
        wh = wh_ref[...]
        cvec = cvec_ref[...]

        def step(k, h):
            t = (tmax - 1 - k) if reverse else k
            for g in range(NG):
                st = starts_ref[0, g]
                ct = counts_ref[0, g]
                safe = jnp.where(t < ct, st + t, 0)
                rw = safe // 8
                sub = safe - rw * 8
                row = hjk_ref[pl.ds(rw, 1), :]
                xg = row[:, 0:JK]
                for o in range(1, 8):
                    xg = jnp.where(sub == o, row[:, o * JK:(o + 1) * JK], xg)
                xt_ref[pl.ds(g, 1), :] = xg
            vmask = t < cvec
            xt = jnp.where(vmask, xt_ref[...], 0.0)
            gi = jnp.dot(xt, wi, preferred_element_type=jnp.float32)
            gh = jnp.dot(h, wh, preferred_element_type=jnp.float32)
            r = jax.nn.sigmoid(gi[:, :RNN] + gh[:, :RNN])
            z = jax.nn.sigmoid(gi[:, RNN:2 * RNN] + gh[:, RNN:2 * RNN])
            n = jnp.tanh(gi[:, 2 * RNN:] + r * gh[:, 2 * RNN:])
            hnew = (1.0 - z) * n + z * h
            hnew = jnp.where(vmask, hnew, h)
            for g in range(NG):
                st = starts_ref[0, g]
                ct = counts_ref[0, g]

                @pl.when(t < ct)
                def _():
                    idx = st + t
                    rw = idx // 8
                    sub = idx - rw * 8
                    csel = sub * 2 + (lane0 // RNN)
                    rowy = y_ref[pl.ds(rw, 1), :]
                    hg = hnew[g:g + 1, :]
                    chunks = [
                        jnp.where(csel == cc, hg,
                                  rowy[:, cc * RNN:(cc + 1) * RNN])
                        for cc in range(16)
                    ]
                    y_ref[pl.ds(rw, 1), :] = jnp.concatenate(chunks, axis=1)

            return hnew

        lax.fori_loop(0, tmax, step, jnp.zeros((NG, RNN), jnp.float32))

    run_dir(wif_ref, whf_ref, 0, False)
    run_dir(wib_ref, whb_ref, RNN, True)


def _tc_gru(starts, counts, tm, cvec, hjk, wif, whf, wib, whb):
    return pl.pallas_call(
        _gru_body,
        in_specs=[
            pl.BlockSpec(memory_space=pltpu.MemorySpace.SMEM),
            pl.BlockSpec(memory_space=pltpu.MemorySpace.SMEM),
            pl.BlockSpec(memory_space=pltpu.MemorySpace.SMEM),
            pl.BlockSpec(memory_space=pltpu.MemorySpace.VMEM),
            pl.BlockSpec(memory_space=pltpu.MemorySpace.VMEM),
            pl.BlockSpec(memory_space=pltpu.MemorySpace.VMEM),
            pl.BlockSpec(memory_space=pltpu.MemorySpace.VMEM),
            pl.BlockSpec(memory_space=pltpu.MemorySpace.VMEM),
            pl.BlockSpec(memory_space=pltpu.MemorySpace.VMEM),
        ],
        out_specs=pl.BlockSpec(memory_space=pltpu.MemorySpace.VMEM),
        out_shape=jax.ShapeDtypeStruct((N // 8, 16 * RNN), jnp.float32),
        scratch_shapes=[pltpu.VMEM((NG, JK), jnp.float32)],
        compiler_params=pltpu.CompilerParams(
            vmem_limit_bytes=120 * 1024 * 1024),
    )(starts, counts, tm, cvec, hjk, wif, whf, wib, whb)


def _mlp_body(y_ref, w1_ref, b1_ref, w2_ref, b2_ref, o_ref):
    z = jnp.dot(y_ref[...], w1_ref[...], preferred_element_type=jnp.float32)
    z = jnp.maximum(z + b1_ref[...], 0.0)
    o_ref[...] = jnp.dot(z, w2_ref[...],
                         preferred_element_type=jnp.float32) + b2_ref[...]


def _tc_mlp(y, w1, b1, w2, b2):
    return pl.pallas_call(
        _mlp_body,
        grid=(N // RBLK,),
        in_specs=[
            pl.BlockSpec((RBLK, 2 * RNN), lambda i: (i, 0)),
            pl.BlockSpec((2 * RNN, 64), lambda i: (0, 0)),
            pl.BlockSpec((1, 64), lambda i: (0, 0)),
            pl.BlockSpec((64, 2), lambda i: (0, 0)),
            pl.BlockSpec((1, 2), lambda i: (0, 0)),
        ],
        out_specs=pl.BlockSpec((RBLK, 2), lambda i: (i, 0)),
        out_shape=jax.ShapeDtypeStruct((N, 2), jnp.float32),
    )(y, w1, b1, w2, b2)


def kernel(x, edge_index, edge_attr, batch, params):
    src = edge_index[0]
    dst = edge_index[1]
    gats = params['gats']

    bounds = jnp.searchsorted(batch, jnp.arange(NG + 1, dtype=jnp.int32),
                              side='left').astype(jnp.int32)
    starts = bounds[:NG]
    counts = bounds[1:] - starts
    tm = jnp.max(counts)

    u_all = jnp.stack([p['lin_edge'] @ p['att_edge'] for p in gats], axis=1)
    w_e = params['edge_w'] @ u_all
    c_e = (params['edge_b'] @ u_all).reshape(1, 3)
    esc = _tc_escore(edge_attr, w_e, c_e)
    esc_t = esc.T

    w0 = params['node_w'] @ gats[0]['lin']
    b0 = (params['node_b'] @ gats[0]['lin']).reshape(1, HID)
    att = [jnp.stack([p['att_src'], p['att_dst']], axis=1) for p in gats]

    hs, s2 = _tc_proj0(x, w0, b0, att[0])
    houts = []
    for l in range(3):
        s_src = s2[:, 0]
        s_dst = s2[:, 1]
        t_e = _sc_src_score(src, esc_t[l], s_src)
        mmh, mdh = _sc_message(src, dst, t_e, s_dst, hs)
        mdh = mdh.reshape(NC, NPH, QNP)
        mm = jnp.concatenate([mmh[:, p, :QR, :] for p in range(NPH)], axis=1)
        md = jnp.concatenate([mdh[:, p, :QR] for p in range(NPH)], axis=1)
        nl = min(l + 1, 2)
        h_l, hs, s2 = _tc_norm(mm, md, gats[l]['bias'].reshape(1, HID),
                               gats[nl]['lin'], att[nl])
        houts.append(h_l)

    hjk = jnp.concatenate(houts, axis=-1).reshape(N // 8, 8 * JK)
    gru = params['gru']
    y = _tc_gru(starts.reshape(1, NG), counts.reshape(1, NG),
                tm.reshape(1, 1), counts.reshape(NG, 1).astype(jnp.int32),
                hjk,
                gru['fw']['w_ih'].T, gru['fw']['w_hh'].T,
                gru['bw']['w_ih'].T, gru['bw']['w_hh'].T).reshape(N, 2 * RNN)
    return _tc_mlp(y, params['h1_w'], params['h1_b'].reshape(1, 64),
                   params['h2_w'], params['h2_b'].reshape(1, 2))


# speedup vs baseline: 3.3401x; 1.7684x over previous
import functools
import jax
import jax.numpy as jnp
from jax import lax
from jax.experimental import pallas as pl
from jax.experimental.pallas import tpu as pltpu
from jax.experimental.pallas import tpu_sc as plsc

N = 100000
E = 3200000
NG = 64
HID = 16
RNN = 32
JK = 48

NC = 2
NS = 16
NW = NC * NS
EPW = E // NW
NP = 100096
RPS = NP // NS
NPH = 8
QR = 12500
QNP = 12544
RPH = QNP // NS
JUNK = 12520

BA = 4000
BB = 400
RBLK = 2000
EBLK = 16000

_mesh = plsc.VectorSubcoreMesh(core_axis_name="c", subcore_axis_name="s")


@functools.partial(
    pl.kernel, mesh=_mesh,
    out_type=jax.ShapeDtypeStruct((E,), jnp.float32),
    scratch_types=[
        pltpu.VMEM((N,), jnp.float32),
        pltpu.VMEM((BA,), jnp.int32),
        pltpu.VMEM((BA,), jnp.float32),
        pltpu.VMEM((BA,), jnp.float32),
    ],
    compiler_params=pltpu.CompilerParams(needs_layout_passes=False),
)
def _sc_src_score(src_hbm, esc_hbm, stab_hbm, t_hbm, tab_v, idx_v, esc_v, t_v):
    c = lax.axis_index("c")
    s = lax.axis_index("s")
    wid = s * NC + c
    base = wid * EPW
    pltpu.sync_copy(stab_hbm, tab_v)

    def blk(b, carry):
        off = base + b * BA
        pltpu.sync_copy(src_hbm.at[pl.ds(off, BA)], idx_v)
        pltpu.sync_copy(esc_hbm.at[pl.ds(off, BA)], esc_v)

        def grp(i, cc):
            ii = i * 16
            idx16 = idx_v[pl.ds(ii, 16)]
            v = plsc.load_gather(tab_v, [idx16])
            t_v[pl.ds(ii, 16)] = v + esc_v[pl.ds(ii, 16)]
            return cc

        lax.fori_loop(0, BA // 16, grp, 0)
        pltpu.sync_copy(t_v, t_hbm.at[pl.ds(off, BA)])
        return carry

    lax.fori_loop(0, EPW // BA, blk, 0)


BLD = 272

@functools.partial(
    pl.kernel, mesh=_mesh,
    out_type=(jax.ShapeDtypeStruct((NC, NP, HID), jnp.float32),
              jax.ShapeDtypeStruct((NC * NP,), jnp.float32),
              jax.ShapeDtypeStruct((NP, HID), jnp.float32),
              jax.ShapeDtypeStruct((NP, HID), jnp.float32)),
    scratch_types=[
        pltpu.VMEM((BB,), jnp.int32),
        pltpu.VMEM((BB,), jnp.int32),
        pltpu.VMEM((BB,), jnp.float32),
        pltpu.VMEM((BB,), jnp.float32),
        pltpu.VMEM((BB, HID), jnp.float32),
        pltpu.VMEM((BLD * HID,), jnp.float32),
        pltpu.VMEM_SHARED((NP, HID), jnp.float32),
        pltpu.VMEM_SHARED((NP,), jnp.float32),
        pltpu.VMEM_SHARED((NP,), jnp.float32),
        pltpu.SemaphoreType.DMA,
        pltpu.SemaphoreType.DMA,
    ],
    compiler_params=pltpu.CompilerParams(needs_layout_passes=False,
                                         use_tc_tiling_on_sc=False),
)
def _sc_message(src_hbm, dst_hbm, t_hbm, dtab_hbm, hsflat_hbm,
                outm_hbm, outd_hbm, tab0_hbm, tab1_hbm,
                src_v, dst_v, t_v, ex_v, rows_v, flat_v, accm, accd, sdtab,
                sem, sem2):
    c = lax.axis_index("c")
    s = lax.axis_index("s")
    wid = s * NC + c
    base = wid * EPW
    r0 = s * RPS
    pltpu.sync_copy(dtab_hbm.at[pl.ds(r0, RPS)], sdtab.at[pl.ds(r0, RPS)])

    def build(tab_hbm):
        done = 0
        while done < RPS:
            step = min(BLD, RPS - done)
            pltpu.sync_copy(
                hsflat_hbm.at[pl.ds((r0 + done) * HID, step * HID)],
                flat_v.at[pl.ds(0, step * HID)])

            def cp(i, cc):
                rows_v[i, :] = flat_v[pl.ds(i * HID, 16)]
                return cc

            lax.fori_loop(0, min(step, BB), cp, 0)
            if step > BB:
                def cp2(i, cc):
                    rows2 = flat_v[pl.ds((BB + i) * HID, 16)]
                    rows_v[i, :] = rows2
                    return cc
                pltpu.sync_copy(rows_v.at[pl.ds(0, BB)],
                                tab_hbm.at[pl.ds(r0 + done, BB)])
                lax.fori_loop(0, step - BB, cp2, 0)
                pltpu.sync_copy(rows_v.at[pl.ds(0, step - BB)],
                                tab_hbm.at[pl.ds(r0 + done + BB, step - BB)])
            else:
                pltpu.sync_copy(rows_v.at[pl.ds(0, step)],
                                tab_hbm.at[pl.ds(r0 + done, step)])
            done += step

    @pl.when(c == 0)
    def _():
        build(tab0_hbm)

    @pl.when(c == 1)
    def _():
        build(tab1_hbm)

    def z16(i, cc):
        rows_v[i, :] = jnp.zeros((16,), jnp.float32)
        return cc

    lax.fori_loop(0, BB, z16, 0)

    def z1(i, cc):
        ex_v[pl.ds(i * 16, 16)] = jnp.zeros((16,), jnp.float32)
        return cc

    lax.fori_loop(0, BB // 16, z1, 0)

    done = 0
    while done < RPS:
        step = min(BB, RPS - done)
        pltpu.sync_copy(rows_v.at[pl.ds(0, step)],
                        accm.at[pl.ds(r0 + done, step)])
        pltpu.sync_copy(ex_v.at[pl.ds(0, step)],
                        accd.at[pl.ds(r0 + done, step)])
        done += step
    plsc.subcore_barrier()

    def body(b, tab_hbm):
        off = base + b * BB
        pltpu.sync_copy(src_hbm.at[pl.ds(off, BB)], src_v)
        pltpu.sync_copy(dst_hbm.at[pl.ds(off, BB)], dst_v)
        pltpu.sync_copy(t_hbm.at[pl.ds(off, BB)], t_v)
        sdg = pltpu.async_copy(sdtab.at[dst_v], ex_v, sem2)
        gat = pltpu.async_copy(tab_hbm.at[src_v], rows_v, sem)
        sdg.wait()

        def grp(i, cc):
            ii = i * 16
            a = t_v[pl.ds(ii, 16)] + ex_v[pl.ds(ii, 16)]
            a = jnp.where(a > 0.0, a, 0.2 * a)
            ex_v[pl.ds(ii, 16)] = jnp.exp(a)
            return cc

        lax.fori_loop(0, BB // 16, grp, 0)
        gat.wait()

        def scale(j, cc):
            jj = j * 16
            ex16 = ex_v[pl.ds(jj, 16)]
            for u in range(16):
                rows_v[jj + u, :] = rows_v[jj + u, :] * ex16[u]
            return cc

        lax.fori_loop(0, BB // 16, scale, 0)
        pltpu.sync_copy(rows_v, accm.at[dst_v], add=True)
        pltpu.sync_copy(ex_v, accd.at[dst_v], add=True)

    @pl.when(c == 0)
    def _():
        lax.fori_loop(0, EPW // BB, lambda b, cc: (body(b, tab0_hbm), cc)[1], 0)

    @pl.when(c == 1)
    def _():
        lax.fori_loop(0, EPW // BB, lambda b, cc: (body(b, tab1_hbm), cc)[1], 0)

    plsc.subcore_barrier()
    pltpu.sync_copy(accm.at[pl.ds(r0, RPS)], outm_hbm.at[c, pl.ds(r0, RPS)])
    pltpu.sync_copy(accd.at[pl.ds(r0, RPS)], outd_hbm.at[pl.ds(c * NP + r0, RPS)])


def _proj0_body(x_ref, w_ref, b_ref, a_ref, hs_ref, s2_ref):
    hs = jnp.dot(x_ref[...], w_ref[...], preferred_element_type=jnp.float32)
    hs = hs + b_ref[...]
    hs_ref[...] = hs
    s2_ref[...] = jnp.dot(hs, a_ref[...], preferred_element_type=jnp.float32)


def _tc_proj0(x, w, b, a):
    return pl.pallas_call(
        _proj0_body,
        grid=(N // RBLK,),
        in_specs=[
            pl.BlockSpec((RBLK, 128), lambda i: (i, 0)),
            pl.BlockSpec((128, HID), lambda i: (0, 0)),
            pl.BlockSpec((1, HID), lambda i: (0, 0)),
            pl.BlockSpec((HID, 2), lambda i: (0, 0)),
        ],
        out_specs=[
            pl.BlockSpec((RBLK, HID), lambda i: (i, 0)),
            pl.BlockSpec((RBLK, 2), lambda i: (i, 0)),
        ],
        out_shape=[
            jax.ShapeDtypeStruct((N, HID), jnp.float32),
            jax.ShapeDtypeStruct((N, 2), jnp.float32),
        ],
    )(x, w, b, a)


def _escore_body(ea_ref, w_ref, c_ref, o_ref):
    o_ref[...] = jnp.dot(ea_ref[...], w_ref[...],
                         preferred_element_type=jnp.float32) + c_ref[...]


def _tc_escore(ea, w, c):
    return pl.pallas_call(
        _escore_body,
        grid=(E // EBLK,),
        in_specs=[
            pl.BlockSpec((EBLK, 16), lambda i: (i, 0)),
            pl.BlockSpec((16, 3), lambda i: (0, 0)),
            pl.BlockSpec((1, 3), lambda i: (0, 0)),
        ],
        out_specs=pl.BlockSpec((EBLK, 3), lambda i: (i, 0)),
        out_shape=jax.ShapeDtypeStruct((E, 3), jnp.float32),
    )(ea, w, c)


def _norm_body(mm_ref, md_ref, bias_ref, lin_ref, a_ref, h_ref, hs_ref, s2_ref):
    m = mm_ref[0] + mm_ref[1]
    d = md_ref[0] + md_ref[1]
    h = m / (d + 1e-16) + bias_ref[...]
    h = jnp.where(h > 0.0, h, 0.01 * h)
    h_ref[...] = h
    hs = jnp.dot(h, lin_ref[...], preferred_element_type=jnp.float32)
    hs_ref[...] = hs
    s2_ref[...] = jnp.dot(hs, a_ref[...], preferred_element_type=jnp.float32)


def _tc_norm(mm, md, bias, lin, a):
    md3 = md.reshape(NC, N, 1)
    return pl.pallas_call(
        _norm_body,
        grid=(N // RBLK,),
        in_specs=[
            pl.BlockSpec((NC, RBLK, HID), lambda i: (0, i, 0)),
            pl.BlockSpec((NC, RBLK, 1), lambda i: (0, i, 0)),
            pl.BlockSpec((1, HID), lambda i: (0, 0)),
            pl.BlockSpec((HID, HID), lambda i: (0, 0)),
            pl.BlockSpec((HID, 2), lambda i: (0, 0)),
        ],
        out_specs=[
            pl.BlockSpec((RBLK, HID), lambda i: (i, 0)),
            pl.BlockSpec((RBLK, HID), lambda i: (i, 0)),
            pl.BlockSpec((RBLK, 2), lambda i: (i, 0)),
        ],
        out_shape=[
            jax.ShapeDtypeStruct((N, HID), jnp.float32),
            jax.ShapeDtypeStruct((N, HID), jnp.float32),
            jax.ShapeDtypeStruct((N, 2), jnp.float32),
        ],
    )(mm, md3, bias, lin, a)


def _gru_body(starts_ref, counts_ref, tm_ref, cvec_ref, hjk_ref,
              wif_ref, whf_ref, wib_ref, whb_ref, y_ref, xt_ref):
    tmax = tm_ref[0, 0]

    def run_dir(wi_ref, wh_ref, lane0, reverse):
        wi = wi_ref[...]
        wh = wh_ref[...]
        cvec = cvec_ref[...]

        def step(k, h):
            t = (tmax - 1 - k) if reverse else k
            for g in range(NG):
                st = starts_ref[0, g]
                ct = counts_ref[0, g]
                safe = jnp.where(t < ct, st + t, 0)
                rw = safe // 8
                sub = safe - rw * 8
                row = hjk_ref[pl.ds(rw, 1), :]
                xg = row[:, 0:JK]
                for o in range(1, 8):
                    xg = jnp.where(sub == o, row[:, o * JK:(o + 1) * JK], xg)
                xt_ref[pl.ds(g, 1), :] = xg
            vmask = t < cvec
            xt = jnp.where(vmask, xt_ref[...], 0.0)
            gi = jnp.dot(xt, wi, preferred_element_type=jnp.float32)
            gh = jnp.dot(h, wh, preferred_element_type=jnp.float32)
            r = jax.nn.sigmoid(gi[:, :RNN] + gh[:, :RNN])
            z = jax.nn.sigmoid(gi[:, RNN:2 * RNN] + gh[:, RNN:2 * RNN])
            n = jnp.tanh(gi[:, 2 * RNN:] + r * gh[:, 2 * RNN:])
            hnew = (1.0 - z) * n + z * h
            hnew = jnp.where(vmask, hnew, h)
            for g in range(NG):
                st = starts_ref[0, g]
                ct = counts_ref[0, g]

                @pl.when(t < ct)
                def _():
                    idx = st + t
                    rw = idx // 8
                    sub = idx - rw * 8
                    csel = sub * 2 + (lane0 // RNN)
                    rowy = y_ref[pl.ds(rw, 1), :]
                    hg = hnew[g:g + 1, :]
                    chunks = [
                        jnp.where(csel == cc, hg,
                                  rowy[:, cc * RNN:(cc + 1) * RNN])
                        for cc in range(16)
                    ]
                    y_ref[pl.ds(rw, 1), :] = jnp.concatenate(chunks, axis=1)

            return hnew

        lax.fori_loop(0, tmax, step, jnp.zeros((NG, RNN), jnp.float32))

    run_dir(wif_ref, whf_ref, 0, False)
    run_dir(wib_ref, whb_ref, RNN, True)


def _tc_gru(starts, counts, tm, cvec, hjk, wif, whf, wib, whb):
    return pl.pallas_call(
        _gru_body,
        in_specs=[
            pl.BlockSpec(memory_space=pltpu.MemorySpace.SMEM),
            pl.BlockSpec(memory_space=pltpu.MemorySpace.SMEM),
            pl.BlockSpec(memory_space=pltpu.MemorySpace.SMEM),
            pl.BlockSpec(memory_space=pltpu.MemorySpace.VMEM),
            pl.BlockSpec(memory_space=pltpu.MemorySpace.VMEM),
            pl.BlockSpec(memory_space=pltpu.MemorySpace.VMEM),
            pl.BlockSpec(memory_space=pltpu.MemorySpace.VMEM),
            pl.BlockSpec(memory_space=pltpu.MemorySpace.VMEM),
            pl.BlockSpec(memory_space=pltpu.MemorySpace.VMEM),
        ],
        out_specs=pl.BlockSpec(memory_space=pltpu.MemorySpace.VMEM),
        out_shape=jax.ShapeDtypeStruct((N // 8, 16 * RNN), jnp.float32),
        scratch_shapes=[pltpu.VMEM((NG, JK), jnp.float32)],
        compiler_params=pltpu.CompilerParams(
            vmem_limit_bytes=120 * 1024 * 1024),
    )(starts, counts, tm, cvec, hjk, wif, whf, wib, whb)


def _mlp_body(y_ref, w1_ref, b1_ref, w2_ref, b2_ref, o_ref):
    z = jnp.dot(y_ref[...], w1_ref[...], preferred_element_type=jnp.float32)
    z = jnp.maximum(z + b1_ref[...], 0.0)
    o_ref[...] = jnp.dot(z, w2_ref[...],
                         preferred_element_type=jnp.float32) + b2_ref[...]


def _tc_mlp(y, w1, b1, w2, b2):
    return pl.pallas_call(
        _mlp_body,
        grid=(N // RBLK,),
        in_specs=[
            pl.BlockSpec((RBLK, 2 * RNN), lambda i: (i, 0)),
            pl.BlockSpec((2 * RNN, 64), lambda i: (0, 0)),
            pl.BlockSpec((1, 64), lambda i: (0, 0)),
            pl.BlockSpec((64, 2), lambda i: (0, 0)),
            pl.BlockSpec((1, 2), lambda i: (0, 0)),
        ],
        out_specs=pl.BlockSpec((RBLK, 2), lambda i: (i, 0)),
        out_shape=jax.ShapeDtypeStruct((N, 2), jnp.float32),
    )(y, w1, b1, w2, b2)


def kernel(x, edge_index, edge_attr, batch, params):
    src = edge_index[0]
    dst = edge_index[1]
    gats = params['gats']

    bounds = jnp.searchsorted(batch, jnp.arange(NG + 1, dtype=jnp.int32),
                              side='left').astype(jnp.int32)
    starts = bounds[:NG]
    counts = bounds[1:] - starts
    tm = jnp.max(counts)

    u_all = jnp.stack([p['lin_edge'] @ p['att_edge'] for p in gats], axis=1)
    w_e = params['edge_w'] @ u_all
    c_e = (params['edge_b'] @ u_all).reshape(1, 3)
    esc = _tc_escore(edge_attr, w_e, c_e)
    esc_t = esc.T

    w0 = params['node_w'] @ gats[0]['lin']
    b0 = (params['node_b'] @ gats[0]['lin']).reshape(1, HID)
    att = [jnp.stack([p['att_src'], p['att_dst']], axis=1) for p in gats]

    hs, s2 = _tc_proj0(x, w0, b0, att[0])
    houts = []
    for l in range(3):
        s_src = s2[:, 0]
        s_dst = s2[:, 1]
        hs_flat = jnp.pad(hs.reshape(N * HID), (0, (NP - N) * HID))
        t_e = _sc_src_score(src, esc_t[l], s_src)
        mmh, mdh, _, _ = _sc_message(src, dst, t_e,
                                     jnp.pad(s_dst, (0, NP - N)), hs_flat)
        mm = mmh[:, :N, :]
        md = mdh.reshape(NC, NP)[:, :N]
        nl = min(l + 1, 2)
        h_l, hs, s2 = _tc_norm(mm, md, gats[l]['bias'].reshape(1, HID),
                               gats[nl]['lin'], att[nl])
        houts.append(h_l)

    hjk = jnp.concatenate(houts, axis=-1).reshape(N // 8, 8 * JK)
    gru = params['gru']
    y = _tc_gru(starts.reshape(1, NG), counts.reshape(1, NG),
                tm.reshape(1, 1), counts.reshape(NG, 1).astype(jnp.int32),
                hjk,
                gru['fw']['w_ih'].T, gru['fw']['w_hh'].T,
                gru['bw']['w_ih'].T, gru['bw']['w_hh'].T).reshape(N, 2 * RNN)
    return _tc_mlp(y, params['h1_w'], params['h1_b'].reshape(1, 64),
                   params['h2_w'], params['h2_b'].reshape(1, 2))
